# baseline (speedup 1.0000x reference)
import functools

import jax
import jax.numpy as jnp
from jax import lax
from jax.experimental import pallas as pl
from jax.experimental.pallas import tpu as pltpu
from jax.experimental.pallas import tpu_sc as plsc

LANES = 16
NBUF = 3
JBLK = 8


def _sc_text_embed(text, table, freqs):
    B, NT = text.shape
    D = table.shape[1]
    info = plsc.get_sparse_core_info()
    NC, NS = info.num_cores, info.num_subcores
    NW = NC * NS
    rows_per_w = B // NW
    assert B % NW == 0 and D % LANES == 0

    NA = 112
    NB_REAL = NT - NA
    NB = ((NB_REAL + LANES - 1) // LANES) * LANES
    NR = NA + NB
    NTOK = rows_per_w * NT
    assert NA % JBLK == 0 and NB_REAL % JBLK == 0 and NT % JBLK == 0
    V = table.shape[0]
    VP = ((V + 7) // 8) * 8

    mesh = plsc.VectorSubcoreMesh(core_axis_name="c", subcore_axis_name="s")

    @functools.partial(
        pl.kernel,
        mesh=mesh,
        out_type=jax.ShapeDtypeStruct((B, NT, D), jnp.float32),
        scratch_types=[
            pltpu.VMEM((NTOK + LANES,), jnp.int32),
            pltpu.VMEM((NBUF, NA), jnp.int32),
            pltpu.VMEM((NBUF, NB), jnp.int32),
            pltpu.VMEM((NT, D), jnp.float32),
            pltpu.VMEM((NBUF, NR, D), jnp.float32),
            pltpu.VMEM_SHARED((VP, D), jnp.float32),
        ]
        + [pltpu.SemaphoreType.DMA] * (3 * NBUF),
    )
    def k(text_hbm, table_hbm, freqs_hbm, out_hbm,
          idx_all, idx_a, idx_b, freqs_v, rows, table_sh, *sems):
        sem_ga = sems[0:NBUF]
        sem_gb = sems[NBUF:2 * NBUF]
        sem_o = sems[2 * NBUF:3 * NBUF]
        wid = lax.axis_index("s") * NC + lax.axis_index("c")
        base = wid * rows_per_w
        tok_base = base * NT

        pltpu.sync_copy(freqs_hbm.at[pl.ds(0, NT)], freqs_v)
        idx_all[pl.ds(NTOK, LANES)] = jnp.zeros((LANES,), jnp.int32)
        pltpu.sync_copy(text_hbm.at[pl.ds(tok_base, NTOK)],
                        idx_all.at[pl.ds(0, NTOK)])

        @pl.when(lax.axis_index("s") == 0)
        def _():
            pltpu.sync_copy(table_hbm, table_sh)
        plsc.subcore_barrier()

        def prep_gather(r, s):
            o = r * NT
            for i in range(NA // LANES):
                idx_a[s, pl.ds(i * LANES, LANES)] = (
                    idx_all[pl.ds(o + i * LANES, LANES)] + 1)
            for i in range(NB // LANES):
                idx_b[s, pl.ds(i * LANES, LANES)] = (
                    idx_all[pl.ds(o + NA + i * LANES, LANES)] + 1)
            pltpu.async_copy(table_sh.at[idx_a.at[s]],
                             rows.at[s, pl.ds(0, NA)], sem_ga[s])
            pltpu.async_copy(table_sh.at[idx_b.at[s]],
                             rows.at[s, pl.ds(NA, NB)], sem_gb[s])

        def add_block(s, lo, hi):
            def add(i, c):
                j8 = i * JBLK
                for jj in range(JBLK):
                    for ch in range(D // LANES):
                        sl = pl.ds(ch * LANES, LANES)
                        plsc.addupdate(rows.at[s, j8 + jj, sl],
                                       freqs_v[j8 + jj, sl])
                return c
            lax.fori_loop(lo // JBLK, hi // JBLK, add, 0)

        def process(r, s):
            b = base + r
            pltpu.make_async_copy(table_sh.at[idx_a.at[s]],
                                  rows.at[s, pl.ds(0, NA)], sem_ga[s]).wait()
            add_block(s, 0, NA)
            pltpu.make_async_copy(table_sh.at[idx_b.at[s]],
                                  rows.at[s, pl.ds(NA, NB)], sem_gb[s]).wait()
            add_block(s, NA, NT)
            pltpu.async_copy(rows.at[s, pl.ds(0, NT)], out_hbm.at[b],
                             sem_o[s])

        def wait_out(r, s):
            b = base + r
            pltpu.make_async_copy(rows.at[s, pl.ds(0, NT)], out_hbm.at[b],
                                  sem_o[s]).wait()

        prep_gather(0, 0)

        def body(kk, c):
            r0 = kk * NBUF
            for d in range(NBUF):
                r = r0 + d
                sn = (d + 1) % NBUF
                if d < NBUF - 1:
                    @pl.when(kk > 0)
                    def _():
                        wait_out(r + 1 - NBUF, sn)
                else:
                    wait_out(r + 1 - NBUF, sn)
                prep_gather(r + 1, sn)
                process(r, d)
            return c

        n_main = (rows_per_w - 2) // NBUF
        assert n_main * NBUF == rows_per_w - 2
        lax.fori_loop(0, n_main, body, 0)

        r30, r31 = rows_per_w - 2, rows_per_w - 1
        wait_out(r30 - 2, (r30 - 2) % NBUF)
        prep_gather(r31, r31 % NBUF)
        process(r30, r30 % NBUF)
        process(r31, r31 % NBUF)
        wait_out(r30 - 1, (r30 - 1) % NBUF)
        wait_out(r30, r30 % NBUF)
        wait_out(r31, r31 % NBUF)

    table_p = jnp.concatenate(
        [table, jnp.zeros((VP - V, D), table.dtype)]) if VP != V else table
    return k(text.reshape(-1), table_p, freqs)


def kernel(text, text_embed_table, freqs_cis):
    return _sc_text_embed(text, text_embed_table, freqs_cis)


# speedup vs baseline: 1.0239x; 1.0239x over previous
import functools

import jax
import jax.numpy as jnp
from jax import lax
from jax.experimental import pallas as pl
from jax.experimental.pallas import tpu as pltpu
from jax.experimental.pallas import tpu_sc as plsc

LANES = 16
NBUF = 3
JBLK = 8


def _sc_text_embed(text, table, freqs):
    B, NT = text.shape
    D = table.shape[1]
    info = plsc.get_sparse_core_info()
    NC, NS = info.num_cores, info.num_subcores
    NW = NC * NS
    rows_per_w = B // NW
    assert B % NW == 0 and D % LANES == 0

    NA = 112
    NB_REAL = NT - NA
    NB = ((NB_REAL + LANES - 1) // LANES) * LANES
    NTOK = rows_per_w * NT
    assert NA % JBLK == 0 and NB_REAL % JBLK == 0
    V = table.shape[0]
    VP = ((V + 7) // 8) * 8

    mesh = plsc.VectorSubcoreMesh(core_axis_name="c", subcore_axis_name="s")

    @functools.partial(
        pl.kernel,
        mesh=mesh,
        out_type=jax.ShapeDtypeStruct((B, NT, D), jnp.float32),
        scratch_types=[
            pltpu.VMEM((NTOK + LANES,), jnp.int32),
            pltpu.VMEM((NBUF, NA), jnp.int32),
            pltpu.VMEM((NBUF, NB), jnp.int32),
            pltpu.VMEM((NT, D), jnp.float32),
            pltpu.VMEM((NBUF, NA, D), jnp.float32),
            pltpu.VMEM((NBUF, NB, D), jnp.float32),
            pltpu.VMEM_SHARED((VP, D), jnp.float32),
        ]
        + [pltpu.SemaphoreType.DMA] * (4 * NBUF),
    )
    def k(text_hbm, table_hbm, freqs_hbm, out_hbm,
          idx_all, idx_a, idx_b, freqs_v, rows_a, rows_b, table_sh, *sems):
        sem_ga = sems[0:NBUF]
        sem_gb = sems[NBUF:2 * NBUF]
        sem_oa = sems[2 * NBUF:3 * NBUF]
        sem_ob = sems[3 * NBUF:4 * NBUF]
        wid = lax.axis_index("s") * NC + lax.axis_index("c")
        base = wid * rows_per_w
        tok_base = base * NT

        pltpu.sync_copy(freqs_hbm.at[pl.ds(0, NT)], freqs_v)
        idx_all[pl.ds(NTOK, LANES)] = jnp.zeros((LANES,), jnp.int32)
        pltpu.sync_copy(text_hbm.at[pl.ds(tok_base, NTOK)],
                        idx_all.at[pl.ds(0, NTOK)])

        @pl.when(lax.axis_index("s") == 0)
        def _():
            pltpu.sync_copy(table_hbm, table_sh)
        plsc.subcore_barrier()

        def prep_gather(r, s):
            o = r * NT
            for i in range(NA // LANES):
                idx_a[s, pl.ds(i * LANES, LANES)] = (
                    idx_all[pl.ds(o + i * LANES, LANES)] + 1)
            for i in range(NB // LANES):
                idx_b[s, pl.ds(i * LANES, LANES)] = (
                    idx_all[pl.ds(o + NA + i * LANES, LANES)] + 1)
            pltpu.async_copy(table_sh.at[idx_a.at[s]], rows_a.at[s],
                             sem_ga[s])
            pltpu.async_copy(table_sh.at[idx_b.at[s]], rows_b.at[s],
                             sem_gb[s])

        def wait_gather(s):
            pltpu.make_async_copy(table_sh.at[idx_a.at[s]], rows_a.at[s],
                                  sem_ga[s]).wait()
            pltpu.make_async_copy(table_sh.at[idx_b.at[s]], rows_b.at[s],
                                  sem_gb[s]).wait()

        def issue_out(r, s):
            b = base + r
            pltpu.async_copy(rows_a.at[s], out_hbm.at[b, pl.ds(0, NA)],
                             sem_oa[s])
            pltpu.async_copy(rows_b.at[s, pl.ds(0, NB_REAL)],
                             out_hbm.at[b, pl.ds(NA, NB_REAL)], sem_ob[s])

        def wait_out(r, s):
            b = base + r
            pltpu.make_async_copy(rows_a.at[s], out_hbm.at[b, pl.ds(0, NA)],
                                  sem_oa[s]).wait()
            pltpu.make_async_copy(rows_b.at[s, pl.ds(0, NB_REAL)],
                                  out_hbm.at[b, pl.ds(NA, NB_REAL)],
                                  sem_ob[s]).wait()

        def add_freqs_a(s):
            def add_a(i, c):
                j8 = i * JBLK
                for jj in range(JBLK):
                    for ch in range(D // LANES):
                        sl = pl.ds(ch * LANES, LANES)
                        plsc.addupdate(rows_a.at[s, j8 + jj, sl],
                                       freqs_v[j8 + jj, sl])
                return c
            lax.fori_loop(0, NA // JBLK, add_a, 0)

        def add_freqs_b(s):
            def add_b(i, c):
                j8 = i * JBLK
                for jj in range(JBLK):
                    for ch in range(D // LANES):
                        sl = pl.ds(ch * LANES, LANES)
                        plsc.addupdate(rows_b.at[s, j8 + jj, sl],
                                       freqs_v[NA + j8 + jj, sl])
                return c
            lax.fori_loop(0, NB_REAL // JBLK, add_b, 0)

        def process(r, s):
            b = base + r
            pltpu.make_async_copy(table_sh.at[idx_a.at[s]], rows_a.at[s],
                                  sem_ga[s]).wait()
            add_freqs_a(s)
            pltpu.async_copy(rows_a.at[s], out_hbm.at[b, pl.ds(0, NA)],
                             sem_oa[s])
            pltpu.make_async_copy(table_sh.at[idx_b.at[s]], rows_b.at[s],
                                  sem_gb[s]).wait()
            add_freqs_b(s)
            pltpu.async_copy(rows_b.at[s, pl.ds(0, NB_REAL)],
                             out_hbm.at[b, pl.ds(NA, NB_REAL)], sem_ob[s])

        prep_gather(0, 0)

        def body(kk, c):
            r0 = kk * NBUF
            for d in range(NBUF):
                r = r0 + d
                sn = (d + 1) % NBUF
                if d < NBUF - 1:
                    @pl.when(kk > 0)
                    def _():
                        wait_out(r + 1 - NBUF, sn)
                else:
                    wait_out(r + 1 - NBUF, sn)
                prep_gather(r + 1, sn)
                process(r, d)
            return c

        n_main = (rows_per_w - 2) // NBUF
        assert n_main * NBUF == rows_per_w - 2
        lax.fori_loop(0, n_main, body, 0)

        r30, r31 = rows_per_w - 2, rows_per_w - 1
        wait_out(r30 - 2, (r30 - 2) % NBUF)
        prep_gather(r31, r31 % NBUF)
        process(r30, r30 % NBUF)
        process(r31, r31 % NBUF)
        wait_out(r30 - 1, (r30 - 1) % NBUF)
        wait_out(r30, r30 % NBUF)
        wait_out(r31, r31 % NBUF)

    table_p = jnp.concatenate(
        [table, jnp.zeros((VP - V, D), table.dtype)]) if VP != V else table
    return k(text.reshape(-1), table_p, freqs)


def kernel(text, text_embed_table, freqs_cis):
    return _sc_text_embed(text, text_embed_table, freqs_cis)


# speedup vs baseline: 1.2979x; 1.2676x over previous
import functools

import jax
import jax.numpy as jnp
from jax import lax
from jax.experimental import pallas as pl
from jax.experimental.pallas import tpu as pltpu
from jax.experimental.pallas import tpu_sc as plsc

LANES = 16
NBUF = 3
UB = 256
UH = UB // 2


def _sc_text_embed(text, table, freqs):
    B, NT = text.shape
    D = table.shape[1]
    info = plsc.get_sparse_core_info()
    NC, NS = info.num_cores, info.num_subcores
    NW = NC * NS
    QB = B // UB
    n_units = NT * QB
    upw = n_units // NW
    assert n_units % NW == 0 and B % UB == 0 and D % LANES == 0
    V = table.shape[0]
    VP = ((V + 7) // 8) * 8
    NFR = 16

    mesh = plsc.VectorSubcoreMesh(core_axis_name="c", subcore_axis_name="s")

    @functools.partial(
        pl.kernel,
        mesh=mesh,
        compiler_params=pltpu.CompilerParams(use_tc_tiling_on_sc=False),
        out_type=jax.ShapeDtypeStruct((B * NT, D), jnp.float32),
        scratch_types=[
            pltpu.VMEM((upw, UB), jnp.int32),
            pltpu.VMEM((2 * NBUF, UH), jnp.int32),
            pltpu.VMEM((2 * NBUF, UH), jnp.int32),
            pltpu.VMEM((UB,), jnp.int32),
            pltpu.VMEM((NFR, D), jnp.float32),
            pltpu.VMEM((NBUF, UB, D), jnp.float32),
            pltpu.VMEM_SHARED((VP, D), jnp.float32),
        ]
        + [pltpu.SemaphoreType.DMA] * (4 * NBUF + 1),
    )
    def k(text_hbm, table_hbm, freqs_hbm, out_hbm,
          ids, idx_c, oidx, ramp, freqs_v, rows, table_sh, *sems):
        sem_ga = sems[0:NBUF]
        sem_gb = sems[NBUF:2 * NBUF]
        sem_oa = sems[2 * NBUF:3 * NBUF]
        sem_ob = sems[3 * NBUF:4 * NBUF]
        sem_pf = sems[4 * NBUF]
        wid = lax.axis_index("s") * NC + lax.axis_index("c")
        g0 = wid * upw

        j_lo = lax.shift_right_logical(g0, 2)
        j_lo8 = (j_lo // 8) * 8
        pltpu.sync_copy(freqs_hbm.at[pl.ds(j_lo8, NFR)], freqs_v)

        for u in range(upw):
            g = g0 + u
            off = lax.shift_right_logical(g, 2) * B + (g & 3) * UB
            pltpu.async_copy(text_hbm.at[pl.ds(off, UB)], ids.at[u], sem_pf)
        for u in range(upw):
            g = g0 + u
            off = lax.shift_right_logical(g, 2) * B + (g & 3) * UB
            pltpu.make_async_copy(text_hbm.at[pl.ds(off, UB)], ids.at[u],
                                  sem_pf).wait()

        iota16 = lax.iota(jnp.int32, LANES)
        for i in range(UB // LANES):
            ramp[pl.ds(i * LANES, LANES)] = (iota16 + i * LANES) * NT

        @pl.when(lax.axis_index("s") == 0)
        def _():
            pltpu.sync_copy(table_hbm, table_sh)
        plsc.subcore_barrier()

        def prep_gather(u, s):
            g = g0 + u
            j = lax.shift_right_logical(g, 2)
            obase = (g & 3) * UB * NT + j
            for h in range(2):
                for i in range(UH // LANES):
                    sl = pl.ds(i * LANES, LANES)
                    src = pl.ds(h * UH + i * LANES, LANES)
                    idx_c[2 * s + h, sl] = ids[u, src] + 1
                    oidx[2 * s + h, sl] = ramp[src] + obase
            pltpu.async_copy(table_sh.at[idx_c.at[2 * s]],
                             rows.at[s, pl.ds(0, UH)], sem_ga[s])
            pltpu.async_copy(table_sh.at[idx_c.at[2 * s + 1]],
                             rows.at[s, pl.ds(UH, UH)], sem_gb[s])

        def add_half(s, h, fvec):
            def add(i, c):
                i8 = h * UH + i * 8
                for ii in range(8):
                    for ch in range(D // LANES):
                        sl = pl.ds(ch * LANES, LANES)
                        plsc.addupdate(rows.at[s, i8 + ii, sl], fvec[ch])
                return c
            lax.fori_loop(0, UH // 8, add, 0)

        def process(u, s):
            g = g0 + u
            j = lax.shift_right_logical(g, 2)
            fvec = [freqs_v[j - j_lo8, pl.ds(ch * LANES, LANES)]
                    for ch in range(D // LANES)]
            pltpu.make_async_copy(table_sh.at[idx_c.at[2 * s]],
                                  rows.at[s, pl.ds(0, UH)], sem_ga[s]).wait()
            add_half(s, 0, fvec)
            pltpu.async_copy(rows.at[s, pl.ds(0, UH)],
                             out_hbm.at[oidx.at[2 * s]], sem_oa[s])
            pltpu.make_async_copy(table_sh.at[idx_c.at[2 * s + 1]],
                                  rows.at[s, pl.ds(UH, UH)], sem_gb[s]).wait()
            add_half(s, 1, fvec)
            pltpu.async_copy(rows.at[s, pl.ds(UH, UH)],
                             out_hbm.at[oidx.at[2 * s + 1]], sem_ob[s])

        def wait_out(s):
            pltpu.make_async_copy(rows.at[s, pl.ds(0, UH)],
                                  out_hbm.at[oidx.at[2 * s]], sem_oa[s]).wait()
            pltpu.make_async_copy(rows.at[s, pl.ds(UH, UH)],
                                  out_hbm.at[oidx.at[2 * s + 1]],
                                  sem_ob[s]).wait()

        prep_gather(0, 0)

        def body(kk, c):
            u0 = kk * NBUF
            for d in range(NBUF):
                u = u0 + d
                sn = (d + 1) % NBUF
                if d < NBUF - 1:
                    @pl.when(kk > 0)
                    def _():
                        wait_out(sn)
                else:
                    wait_out(sn)
                prep_gather(u + 1, sn)
                process(u, d)
            return c

        n_main = (upw - 1) // NBUF
        assert n_main * NBUF == upw - 1
        lax.fori_loop(0, n_main, body, 0)

        u_last = upw - 1
        process(u_last, u_last % NBUF)
        wait_out((u_last - 2) % NBUF)
        wait_out((u_last - 1) % NBUF)
        wait_out(u_last % NBUF)

    table_p = jnp.concatenate(
        [table, jnp.zeros((VP - V, D), table.dtype)]) if VP != V else table
    out2d = k(text.T.reshape(-1), table_p, freqs)
    return out2d.reshape(B, NT, D)


def kernel(text, text_embed_table, freqs_cis):
    return _sc_text_embed(text, text_embed_table, freqs_cis)
